# baseline (speedup 1.0000x reference)
import functools

import jax
import jax.numpy as jnp
from jax import lax
from jax.experimental import pallas as pl
from jax.experimental.pallas import tpu as pltpu
from jax.experimental.pallas import tpu_sc as plsc

BATCH = 4096
HIST = 200
D = 64
DP = 128
B = BATCH * HIST
NROWS = 1000001
NROWS_PAD = 1000008

NC = 2
NS = 16
NW = NC * NS
PER_W = B // NW
CHUNK = 128
NCHUNK = PER_W // CHUNK
GROUP = 4
GROUP_ROWS = GROUP * CHUNK
NGROUP = NCHUNK // GROUP

TBLK = 512
HALF = 977 * TBLK
NBLK = HALF // TBLK


def _transpose_body(ta_ref, tb_ref, out_ref):
    out_ref[:, :D] = ta_ref[...].T
    out_ref[:, D:] = tb_ref[...].T


_transpose_call = pl.pallas_call(
    _transpose_body,
    grid=(NBLK,),
    in_specs=[
        pl.BlockSpec((D, TBLK), lambda i: (0, i)),
        pl.BlockSpec((D, TBLK), lambda i: (0, i + NBLK)),
    ],
    out_specs=pl.BlockSpec((TBLK, DP), lambda i: (i, 0)),
    out_shape=jax.ShapeDtypeStruct((HALF, DP), jnp.float32),
)


def _emb_body(idx_hbm, table_hbm, out_hbm, idx_v, rows0, rows1,
              gsem0, gsem1, ssem0, ssem1):
    wid = lax.axis_index("s") * NC + lax.axis_index("c")
    base = wid * PER_W
    pltpu.sync_copy(idx_hbm.at[wid], idx_v)

    def gather(g, buf, sem):
        return [
            pltpu.make_async_copy(
                table_hbm.at[idx_v.at[g * GROUP + k]],
                buf.at[pl.ds(k * CHUNK, CHUNK)],
                sem,
            )
            for k in range(GROUP)
        ]

    def store(g, buf, sem):
        return pltpu.make_async_copy(
            buf,
            out_hbm.at[pl.ds(base + g * GROUP_ROWS, GROUP_ROWS), pl.ds(0, D)],
            sem,
        )

    for c in gather(0, rows0, gsem0):
        c.start()

    def body(gi, carry):
        g0 = 2 * gi
        g1 = g0 + 1

        @pl.when(gi > 0)
        def _():
            store(g1 - 2, rows1, ssem1).wait()

        for c in gather(g1, rows1, gsem1):
            c.start()

        for c in gather(g0, rows0, gsem0):
            c.wait()
        store(g0, rows0, ssem0).start()

        @pl.when(gi < NGROUP // 2 - 1)
        def _():
            store(g0, rows0, ssem0).wait()
            for c in gather(g0 + 2, rows0, gsem0):
                c.start()

        for c in gather(g1, rows1, gsem1):
            c.wait()
        store(g1, rows1, ssem1).start()
        return carry

    lax.fori_loop(0, NGROUP // 2, body, 0)
    store(NGROUP - 2, rows0, ssem0).wait()
    store(NGROUP - 1, rows1, ssem1).wait()


_emb_call = functools.partial(
    pl.kernel,
    mesh=plsc.VectorSubcoreMesh(core_axis_name="c", subcore_axis_name="s"),
    out_type=jax.ShapeDtypeStruct((B, DP), jnp.float32),
    scratch_types=[
        pltpu.VMEM((NCHUNK, CHUNK), jnp.int32),
        pltpu.VMEM((GROUP_ROWS, D), jnp.float32),
        pltpu.VMEM((GROUP_ROWS, D), jnp.float32),
        pltpu.SemaphoreType.DMA,
        pltpu.SemaphoreType.DMA,
        pltpu.SemaphoreType.DMA,
        pltpu.SemaphoreType.DMA,
    ],
    compiler_params=pltpu.CompilerParams(use_tc_tiling_on_sc=False),
)(_emb_body)


def kernel(inputs, emb_edges_weight):
    idx = inputs.reshape(NW, NCHUNK, CHUNK).astype(jnp.int32)
    idx2 = jnp.where(idx < HALF, 2 * idx, 2 * (idx - HALF) + 1)
    tt = emb_edges_weight.T
    table128 = _transpose_call(tt, tt)
    table64 = table128.reshape(2 * HALF, D)
    out = _emb_call(idx2, table64)
    return out[:, :D].reshape(BATCH, HIST, D)


# speedup vs baseline: 1.7105x; 1.7105x over previous
import functools

import jax
import jax.numpy as jnp
from jax import lax
from jax.experimental import pallas as pl
from jax.experimental.pallas import tpu as pltpu
from jax.experimental.pallas import tpu_sc as plsc

BATCH = 4096
HIST = 200
D = 64
DP = 128
B = BATCH * HIST
NROWS = 1000001
NROWS_PAD = 1000008

NC = 2
NS = 16
NW = NC * NS
PER_W = B // NW
CHUNK = 128
NCHUNK = PER_W // CHUNK
GROUP = 4
GROUP_ROWS = GROUP * CHUNK
NGROUP = NCHUNK // GROUP

TBLK = 4096
NBLK = 123
HALF = NBLK * TBLK
_BFLOOR = (NROWS - 1) // TBLK


def _transpose_body(ta_ref, tb_ref, out_ref):
    out_ref[:, :D] = ta_ref[...].T
    out_ref[:, D:] = tb_ref[...].T


_transpose_call = pl.pallas_call(
    _transpose_body,
    grid=(NBLK,),
    in_specs=[
        pl.BlockSpec((D, TBLK), lambda i: (0, i)),
        pl.BlockSpec((D, TBLK), lambda i: (0, jnp.minimum(i + NBLK, _BFLOOR))),
    ],
    out_specs=pl.BlockSpec((TBLK, DP), lambda i: (i, 0)),
    out_shape=jax.ShapeDtypeStruct((HALF, DP), jnp.float32),
)


def _emb_body(idx_hbm, table_hbm, out_hbm, idx_v, rows0, rows1,
              gsem0, gsem1, ssem0, ssem1):
    wid = lax.axis_index("s") * NC + lax.axis_index("c")
    base = wid * PER_W
    pltpu.sync_copy(idx_hbm.at[wid], idx_v)

    def gather(g, buf, sem):
        return [
            pltpu.make_async_copy(
                table_hbm.at[idx_v.at[g * GROUP + k]],
                buf.at[pl.ds(k * CHUNK, CHUNK)],
                sem,
            )
            for k in range(GROUP)
        ]

    def store(g, buf, sem):
        return pltpu.make_async_copy(
            buf,
            out_hbm.at[pl.ds(base + g * GROUP_ROWS, GROUP_ROWS), pl.ds(0, D)],
            sem,
        )

    for c in gather(0, rows0, gsem0):
        c.start()

    def body(gi, carry):
        g0 = 2 * gi
        g1 = g0 + 1

        @pl.when(gi > 0)
        def _():
            store(g1 - 2, rows1, ssem1).wait()

        for c in gather(g1, rows1, gsem1):
            c.start()

        for c in gather(g0, rows0, gsem0):
            c.wait()
        store(g0, rows0, ssem0).start()

        @pl.when(gi < NGROUP // 2 - 1)
        def _():
            store(g0, rows0, ssem0).wait()
            for c in gather(g0 + 2, rows0, gsem0):
                c.start()

        for c in gather(g1, rows1, gsem1):
            c.wait()
        store(g1, rows1, ssem1).start()
        return carry

    lax.fori_loop(0, NGROUP // 2, body, 0)
    store(NGROUP - 2, rows0, ssem0).wait()
    store(NGROUP - 1, rows1, ssem1).wait()


_emb_call = functools.partial(
    pl.kernel,
    mesh=plsc.VectorSubcoreMesh(core_axis_name="c", subcore_axis_name="s"),
    out_type=jax.ShapeDtypeStruct((B, DP), jnp.float32),
    scratch_types=[
        pltpu.VMEM((NCHUNK, CHUNK), jnp.int32),
        pltpu.VMEM((GROUP_ROWS, D), jnp.float32),
        pltpu.VMEM((GROUP_ROWS, D), jnp.float32),
        pltpu.SemaphoreType.DMA,
        pltpu.SemaphoreType.DMA,
        pltpu.SemaphoreType.DMA,
        pltpu.SemaphoreType.DMA,
    ],
    compiler_params=pltpu.CompilerParams(use_tc_tiling_on_sc=False),
)(_emb_body)


def kernel(inputs, emb_edges_weight):
    idx = inputs.reshape(NW, NCHUNK, CHUNK).astype(jnp.int32)
    idx2 = jnp.where(idx < HALF, 2 * idx, 2 * (idx - HALF) + 1)
    tt = emb_edges_weight.T
    table128 = _transpose_call(tt, tt)
    table64 = table128.reshape(2 * HALF, D)
    out = _emb_call(idx2, table64)
    return out[:, :D].reshape(BATCH, HIST, D)


# speedup vs baseline: 1.8078x; 1.0569x over previous
import functools

import jax
import jax.numpy as jnp
from jax import lax
from jax.experimental import pallas as pl
from jax.experimental.pallas import tpu as pltpu
from jax.experimental.pallas import tpu_sc as plsc

BATCH = 4096
HIST = 200
D = 64
DP = 128
B = BATCH * HIST
NROWS = 1000001
NROWS_PAD = 1000008

NC = 2
NS = 16
NW = NC * NS
PER_W = B // NW
CHUNK = 128
NCHUNK = PER_W // CHUNK
GROUP = 4
GROUP_ROWS = GROUP * CHUNK
NGROUP = NCHUNK // GROUP

TBLK = 8192
NBLK = 62
HALF = NBLK * TBLK
_BFLOOR = (NROWS - 1) // TBLK


def _transpose_body(ta_ref, tb_ref, out_ref):
    out_ref[:, :D] = ta_ref[...].T
    out_ref[:, D:] = tb_ref[...].T


_transpose_call = pl.pallas_call(
    _transpose_body,
    grid=(NBLK,),
    in_specs=[
        pl.BlockSpec((D, TBLK), lambda i: (0, i)),
        pl.BlockSpec((D, TBLK), lambda i: (0, jnp.minimum(i + NBLK, _BFLOOR))),
    ],
    out_specs=pl.BlockSpec((TBLK, DP), lambda i: (i, 0)),
    out_shape=jax.ShapeDtypeStruct((HALF, DP), jnp.float32),
)


def _emb_body(idx_hbm, table_hbm, out_hbm, idx_v, rows0, rows1,
              gsem0, gsem1, ssem0, ssem1):
    wid = lax.axis_index("s") * NC + lax.axis_index("c")
    base = wid * PER_W
    pltpu.sync_copy(idx_hbm.at[wid], idx_v)

    def gather(g, buf, sem):
        return [
            pltpu.make_async_copy(
                table_hbm.at[idx_v.at[g * GROUP + k]],
                buf.at[pl.ds(k * CHUNK, CHUNK)],
                sem,
            )
            for k in range(GROUP)
        ]

    def store(g, buf, sem):
        return pltpu.make_async_copy(
            buf,
            out_hbm.at[pl.ds(base + g * GROUP_ROWS, GROUP_ROWS), pl.ds(0, D)],
            sem,
        )

    for c in gather(0, rows0, gsem0):
        c.start()

    def body(gi, carry):
        g0 = 2 * gi
        g1 = g0 + 1

        @pl.when(gi > 0)
        def _():
            store(g1 - 2, rows1, ssem1).wait()

        for c in gather(g1, rows1, gsem1):
            c.start()

        for c in gather(g0, rows0, gsem0):
            c.wait()
        store(g0, rows0, ssem0).start()

        @pl.when(gi < NGROUP // 2 - 1)
        def _():
            store(g0, rows0, ssem0).wait()
            for c in gather(g0 + 2, rows0, gsem0):
                c.start()

        for c in gather(g1, rows1, gsem1):
            c.wait()
        store(g1, rows1, ssem1).start()
        return carry

    lax.fori_loop(0, NGROUP // 2, body, 0)
    store(NGROUP - 2, rows0, ssem0).wait()
    store(NGROUP - 1, rows1, ssem1).wait()


_emb_call = functools.partial(
    pl.kernel,
    mesh=plsc.VectorSubcoreMesh(core_axis_name="c", subcore_axis_name="s"),
    out_type=jax.ShapeDtypeStruct((B, DP), jnp.float32),
    scratch_types=[
        pltpu.VMEM((NCHUNK, CHUNK), jnp.int32),
        pltpu.VMEM((GROUP_ROWS, D), jnp.float32),
        pltpu.VMEM((GROUP_ROWS, D), jnp.float32),
        pltpu.SemaphoreType.DMA,
        pltpu.SemaphoreType.DMA,
        pltpu.SemaphoreType.DMA,
        pltpu.SemaphoreType.DMA,
    ],
    compiler_params=pltpu.CompilerParams(use_tc_tiling_on_sc=False),
)(_emb_body)


def kernel(inputs, emb_edges_weight):
    idx = inputs.reshape(NW, NCHUNK, CHUNK).astype(jnp.int32)
    idx2 = jnp.where(idx < HALF, 2 * idx, 2 * (idx - HALF) + 1)
    tt = emb_edges_weight.T
    table128 = _transpose_call(tt, tt)
    table64 = table128.reshape(2 * HALF, D)
    out = _emb_call(idx2, table64)
    return out[:, :D].reshape(BATCH, HIST, D)


# speedup vs baseline: 1.8498x; 1.0232x over previous
import functools

import jax
import jax.numpy as jnp
from jax import lax
from jax.experimental import pallas as pl
from jax.experimental.pallas import tpu as pltpu
from jax.experimental.pallas import tpu_sc as plsc

BATCH = 4096
HIST = 200
D = 64
DP = 128
B = BATCH * HIST
NROWS = 1000001
NROWS_PAD = 1000008

NC = 2
NS = 16
NW = NC * NS
PER_W = B // NW
CHUNK = 128
NCHUNK = PER_W // CHUNK
GROUP = 4
GROUP_ROWS = GROUP * CHUNK
NGROUP = NCHUNK // GROUP

TBLK = 16384
NBLK = 31
HALF = NBLK * TBLK
_BFLOOR = (NROWS - 1) // TBLK


def _transpose_body(ta_ref, tb_ref, out_ref):
    out_ref[:, :D] = ta_ref[...].T
    out_ref[:, D:] = tb_ref[...].T


_transpose_call = pl.pallas_call(
    _transpose_body,
    grid=(NBLK,),
    in_specs=[
        pl.BlockSpec((D, TBLK), lambda i: (0, i)),
        pl.BlockSpec((D, TBLK), lambda i: (0, jnp.minimum(i + NBLK, _BFLOOR))),
    ],
    out_specs=pl.BlockSpec((TBLK, DP), lambda i: (i, 0)),
    out_shape=jax.ShapeDtypeStruct((HALF, DP), jnp.float32),
)


def _emb_body(idx_hbm, table_hbm, out_hbm, idx_v, rows0, rows1,
              gsem0, gsem1, ssem0, ssem1):
    wid = lax.axis_index("s") * NC + lax.axis_index("c")
    base = wid * PER_W
    pltpu.sync_copy(idx_hbm.at[wid], idx_v)

    def gather(g, buf, sem):
        return [
            pltpu.make_async_copy(
                table_hbm.at[idx_v.at[g * GROUP + k]],
                buf.at[pl.ds(k * CHUNK, CHUNK)],
                sem,
            )
            for k in range(GROUP)
        ]

    def store(g, buf, sem):
        return pltpu.make_async_copy(
            buf,
            out_hbm.at[pl.ds(base + g * GROUP_ROWS, GROUP_ROWS), pl.ds(0, D)],
            sem,
        )

    for c in gather(0, rows0, gsem0):
        c.start()

    def body(gi, carry):
        g0 = 2 * gi
        g1 = g0 + 1

        @pl.when(gi > 0)
        def _():
            store(g1 - 2, rows1, ssem1).wait()

        for c in gather(g1, rows1, gsem1):
            c.start()

        for c in gather(g0, rows0, gsem0):
            c.wait()
        store(g0, rows0, ssem0).start()

        @pl.when(gi < NGROUP // 2 - 1)
        def _():
            store(g0, rows0, ssem0).wait()
            for c in gather(g0 + 2, rows0, gsem0):
                c.start()

        for c in gather(g1, rows1, gsem1):
            c.wait()
        store(g1, rows1, ssem1).start()
        return carry

    lax.fori_loop(0, NGROUP // 2, body, 0)
    store(NGROUP - 2, rows0, ssem0).wait()
    store(NGROUP - 1, rows1, ssem1).wait()


_emb_call = functools.partial(
    pl.kernel,
    mesh=plsc.VectorSubcoreMesh(core_axis_name="c", subcore_axis_name="s"),
    out_type=jax.ShapeDtypeStruct((B, DP), jnp.float32),
    scratch_types=[
        pltpu.VMEM((NCHUNK, CHUNK), jnp.int32),
        pltpu.VMEM((GROUP_ROWS, D), jnp.float32),
        pltpu.VMEM((GROUP_ROWS, D), jnp.float32),
        pltpu.SemaphoreType.DMA,
        pltpu.SemaphoreType.DMA,
        pltpu.SemaphoreType.DMA,
        pltpu.SemaphoreType.DMA,
    ],
    compiler_params=pltpu.CompilerParams(use_tc_tiling_on_sc=False),
)(_emb_body)


def kernel(inputs, emb_edges_weight):
    idx = inputs.reshape(NW, NCHUNK, CHUNK).astype(jnp.int32)
    idx2 = jnp.where(idx < HALF, 2 * idx, 2 * (idx - HALF) + 1)
    tt = emb_edges_weight.T
    table128 = _transpose_call(tt, tt)
    table64 = table128.reshape(2 * HALF, D)
    out = _emb_call(idx2, table64)
    return out[:, :D].reshape(BATCH, HIST, D)


# speedup vs baseline: 1.8561x; 1.0034x over previous
import functools

import jax
import jax.numpy as jnp
from jax import lax
from jax.experimental import pallas as pl
from jax.experimental.pallas import tpu as pltpu
from jax.experimental.pallas import tpu_sc as plsc

BATCH = 4096
HIST = 200
D = 64
DP = 128
B = BATCH * HIST
NROWS = 1000001
NROWS_PAD = 1000008

NC = 2
NS = 16
NW = NC * NS
PER_W = B // NW
CHUNK = 128
NCHUNK = PER_W // CHUNK
GROUP = 5
GROUP_ROWS = GROUP * CHUNK
NGROUP = NCHUNK // GROUP

TBLK = 16384
NBLK = 31
HALF = NBLK * TBLK
_BFLOOR = (NROWS - 1) // TBLK


def _transpose_body(ta_ref, tb_ref, out_ref):
    out_ref[:, :D] = ta_ref[...].T
    out_ref[:, D:] = tb_ref[...].T


_transpose_call = pl.pallas_call(
    _transpose_body,
    grid=(NBLK,),
    in_specs=[
        pl.BlockSpec((D, TBLK), lambda i: (0, i)),
        pl.BlockSpec((D, TBLK), lambda i: (0, jnp.minimum(i + NBLK, _BFLOOR))),
    ],
    out_specs=pl.BlockSpec((TBLK, DP), lambda i: (i, 0)),
    out_shape=jax.ShapeDtypeStruct((HALF, DP), jnp.float32),
)


def _emb_body(idx_hbm, table_hbm, out_hbm, idx_v, rows0, rows1,
              gsem0, gsem1, ssem0, ssem1):
    wid = lax.axis_index("s") * NC + lax.axis_index("c")
    base = wid * PER_W
    pltpu.sync_copy(idx_hbm.at[wid], idx_v)

    def gather(g, buf, sem):
        return [
            pltpu.make_async_copy(
                table_hbm.at[idx_v.at[g * GROUP + k]],
                buf.at[pl.ds(k * CHUNK, CHUNK)],
                sem,
            )
            for k in range(GROUP)
        ]

    def store(g, buf, sem):
        return pltpu.make_async_copy(
            buf,
            out_hbm.at[pl.ds(base + g * GROUP_ROWS, GROUP_ROWS), pl.ds(0, D)],
            sem,
        )

    for c in gather(0, rows0, gsem0):
        c.start()

    def body(gi, carry):
        g0 = 2 * gi
        g1 = g0 + 1

        @pl.when(gi > 0)
        def _():
            store(g1 - 2, rows1, ssem1).wait()

        for c in gather(g1, rows1, gsem1):
            c.start()

        for c in gather(g0, rows0, gsem0):
            c.wait()
        store(g0, rows0, ssem0).start()

        @pl.when(gi < NGROUP // 2 - 1)
        def _():
            store(g0, rows0, ssem0).wait()
            for c in gather(g0 + 2, rows0, gsem0):
                c.start()

        for c in gather(g1, rows1, gsem1):
            c.wait()
        store(g1, rows1, ssem1).start()
        return carry

    lax.fori_loop(0, NGROUP // 2, body, 0)
    store(NGROUP - 2, rows0, ssem0).wait()
    store(NGROUP - 1, rows1, ssem1).wait()


_emb_call = functools.partial(
    pl.kernel,
    mesh=plsc.VectorSubcoreMesh(core_axis_name="c", subcore_axis_name="s"),
    out_type=jax.ShapeDtypeStruct((B, DP), jnp.float32),
    scratch_types=[
        pltpu.VMEM((NCHUNK, CHUNK), jnp.int32),
        pltpu.VMEM((GROUP_ROWS, D), jnp.float32),
        pltpu.VMEM((GROUP_ROWS, D), jnp.float32),
        pltpu.SemaphoreType.DMA,
        pltpu.SemaphoreType.DMA,
        pltpu.SemaphoreType.DMA,
        pltpu.SemaphoreType.DMA,
    ],
    compiler_params=pltpu.CompilerParams(use_tc_tiling_on_sc=False),
)(_emb_body)


def kernel(inputs, emb_edges_weight):
    idx = inputs.reshape(NW, NCHUNK, CHUNK).astype(jnp.int32)
    idx2 = jnp.where(idx < HALF, 2 * idx, 2 * (idx - HALF) + 1)
    tt = emb_edges_weight.T
    table128 = _transpose_call(tt, tt)
    table64 = table128.reshape(2 * HALF, D)
    out = _emb_call(idx2, table64)
    return out[:, :D].reshape(BATCH, HIST, D)
